# baseline (speedup 1.0000x reference)
import jax
import jax.numpy as jnp
from jax import lax
from jax.experimental import pallas as pl
from jax.experimental.pallas import tpu as pltpu
from jax.experimental.pallas import tpu_sc as plsc

B = 16
T = 4096
D = 1024
LANES = 16
WPR = T // 4


def _body(maskw_hbm, x_hbm, out_hbm, mrow, row):
    c = lax.axis_index("c")
    s = lax.axis_index("s")
    wid = s * 2 + c

    @pl.when(wid < B)
    def _():
        pltpu.sync_copy(maskw_hbm.at[pl.ds(wid * WPR, WPR)], mrow)

        def step(i, acc):
            return acc + mrow[pl.ds(i * LANES, LANES)]

        acc = lax.fori_loop(
            0, WPR // LANES, step, jnp.zeros((LANES,), jnp.int32), unroll=8
        )
        bytes_sum = (
            (acc & 0xFF)
            + ((acc >> 8) & 0xFF)
            + ((acc >> 16) & 0xFF)
            + ((acc >> 24) & 0xFF)
        )
        cnt = plsc.cumsum(bytes_sum)[LANES - 1]
        idx = jnp.where(cnt > 0, cnt - 1, T - 1)
        pltpu.sync_copy(x_hbm.at[pl.ds(wid * T + idx, 1)], row)
        pltpu.sync_copy(row, out_hbm.at[pl.ds(wid, 1)])


@jax.jit
def _last_timestamp(x2d, maskw):
    return pl.kernel(
        _body,
        out_type=jax.ShapeDtypeStruct((B, D), jnp.float32),
        mesh=plsc.VectorSubcoreMesh(core_axis_name="c", subcore_axis_name="s"),
        compiler_params=pltpu.CompilerParams(needs_layout_passes=False),
        scratch_types=[
            pltpu.VMEM((WPR,), jnp.int32),
            pltpu.VMEM((1, D), jnp.float32),
        ],
    )(maskw, x2d)


def kernel(x, mask):
    x2d = x.reshape(B * T, D)
    maskw = lax.bitcast_convert_type(
        mask.view(jnp.uint8).reshape(B * WPR, 4), jnp.int32
    )
    return _last_timestamp(x2d, maskw)


# speedup vs baseline: 1.6806x; 1.6806x over previous
import jax
import jax.numpy as jnp
from jax import lax
from jax.experimental import pallas as pl
from jax.experimental.pallas import tpu as pltpu
from jax.experimental.pallas import tpu_sc as plsc

B = 16
T = 4096
D = 1024
LANES = 16


def _body(mask_hbm, x_hbm, out_hbm, mrow, row):
    c = lax.axis_index("c")
    s = lax.axis_index("s")
    wid = s * 2 + c

    @pl.when(wid < B)
    def _():
        pltpu.sync_copy(mask_hbm.at[pl.ds(wid * T, T)], mrow)

        def step(i, acc):
            return acc + mrow[pl.ds(i * LANES, LANES)]

        acc = lax.fori_loop(
            0, T // LANES, step, jnp.zeros((LANES,), jnp.int32), unroll=8
        )
        cnt = plsc.cumsum(acc)[LANES - 1]
        idx = jnp.where(cnt > 0, cnt - 1, T - 1)
        pltpu.sync_copy(x_hbm.at[pl.ds(wid * T + idx, 1)], row)
        pltpu.sync_copy(row, out_hbm.at[pl.ds(wid, 1)])


@jax.jit
def _last_timestamp(x2d, mask1d):
    return pl.kernel(
        _body,
        out_type=jax.ShapeDtypeStruct((B, D), jnp.float32),
        mesh=plsc.VectorSubcoreMesh(core_axis_name="c", subcore_axis_name="s"),
        compiler_params=pltpu.CompilerParams(needs_layout_passes=False),
        scratch_types=[
            pltpu.VMEM((T,), jnp.int32),
            pltpu.VMEM((1, D), jnp.float32),
        ],
    )(mask1d, x2d)


def kernel(x, mask):
    x2d = x.reshape(B * T, D)
    mask1d = mask.astype(jnp.int32).reshape(B * T)
    return _last_timestamp(x2d, mask1d)
